# Initial kernel scaffold; baseline (speedup 1.0000x reference)
import jax
import jax.numpy as jnp
from jax.experimental import pallas as pl


def kernel(hidden_states, attention_mask, merged_embeddings_counts, residual_hidden_states, residual_attention_mask):
    raise NotImplementedError("write your pallas kernel here")


# speedup vs baseline: 3.0804x; 3.0804x over previous
import functools

import jax
import jax.numpy as jnp
from jax import lax
from jax.experimental import pallas as pl
from jax.experimental.pallas import tpu as pltpu
from jax.experimental.pallas import tpu_sc as plsc

B, S, H = 8, 4096, 1024
NC, NS = 2, 16
ROWS_PER_TILE = S // NS
COLS = 128
CHUNKS_PER_CORE = H // COLS // NC
TRASH = 8
CHUNK_ROWS = S + TRASH


def _dst_body(counts_ref, dst_ref):
    c = counts_ref[...]
    bad = (c <= 0).astype(jnp.int32)
    x = jnp.concatenate([c, bad], axis=0)
    k = 1
    while k < S:
        x = x + jnp.concatenate(
            [jnp.zeros((2 * B, k), jnp.int32), x[:, : S - k]], axis=1)
        k *= 2
    cum = x[:B]
    valid = x[B:] == 0
    idx = jnp.clip(cum - 1, 0, S - 1)
    lane = lax.broadcasted_iota(jnp.int32, (B, S), 1)
    dst_ref[...] = jnp.where(valid, idx, S + (lane & (TRASH - 1)))


_dst_rows = pl.pallas_call(
    _dst_body,
    out_shape=jax.ShapeDtypeStruct((B, S), jnp.int32),
)


def _sc_body(hidden, residual, dst3, out, chunk, buf, idxbuf, sem_r, sem_h):
    cid = lax.axis_index("c")
    sid = lax.axis_index("s")
    r0 = pl.multiple_of(sid * ROWS_PER_TILE, ROWS_PER_TILE)
    for b in range(B):
        for cc in range(CHUNKS_PER_CORE):
            col0 = pl.multiple_of((cid * CHUNKS_PER_CORE + cc) * COLS, COLS)
            pltpu.sync_copy(dst3.at[b, pl.ds(sid * 2, 2), :], idxbuf)
            cp_r = pltpu.async_copy(
                residual.at[b, pl.ds(r0, ROWS_PER_TILE), pl.ds(col0, COLS)],
                chunk.at[pl.ds(r0, ROWS_PER_TILE), :], sem_r)
            cp_h = pltpu.async_copy(
                hidden.at[b, pl.ds(r0, ROWS_PER_TILE), pl.ds(col0, COLS)],
                buf, sem_h)
            cp_r.wait()
            cp_h.wait()
            plsc.subcore_barrier()
            for j in range(ROWS_PER_TILE // 128):
                pltpu.sync_copy(buf.at[pl.ds(j * 128, 128), :],
                                chunk.at[idxbuf.at[j]], add=True)
            plsc.subcore_barrier()
            pltpu.sync_copy(
                chunk.at[pl.ds(r0, ROWS_PER_TILE), :],
                out.at[b, pl.ds(r0, ROWS_PER_TILE), pl.ds(col0, COLS)])


_scatter = functools.partial(
    pl.kernel,
    out_type=jax.ShapeDtypeStruct((B, S, H), jnp.float32),
    mesh=plsc.VectorSubcoreMesh(core_axis_name="c", subcore_axis_name="s"),
    scratch_types=[
        pltpu.VMEM_SHARED((CHUNK_ROWS, COLS), jnp.float32),
        pltpu.VMEM((ROWS_PER_TILE, COLS), jnp.float32),
        pltpu.VMEM((ROWS_PER_TILE // 128, 128), jnp.int32),
        pltpu.SemaphoreType.DMA,
        pltpu.SemaphoreType.DMA,
    ],
)(_sc_body)


def kernel(hidden_states, attention_mask, merged_embeddings_counts,
           residual_hidden_states, residual_attention_mask):
    del attention_mask, residual_attention_mask
    dst = _dst_rows(merged_embeddings_counts.astype(jnp.int32))
    dst3 = dst.reshape(B, S // 128, 128)
    return _scatter(hidden_states, residual_hidden_states, dst3)
